# baseline (speedup 1.0000x reference)
import functools

import jax
import jax.numpy as jnp
from jax import lax
from jax.experimental import pallas as pl
from jax.experimental.pallas import tpu as pltpu
from jax.experimental.pallas import tpu_sc as plsc

_NC = 2
_NS = 16
_NW = _NC * _NS
_L = 16
_K = 4
_P = 2
_NI = 4
_NO = 4


def _body(batch, n, x_hbm, perm_hbm, y_hbm, ld_hbm,
          perm_v, in0, in1, in2, in3, out0, out1, out2, out3, zv,
          si0, si1, si2, si3, so0, so1, so2, so3):
    rows_per_tile = batch // _NW
    nch = rows_per_tile // _K
    halfn = n // _P
    nj = halfn // _L
    cid = lax.axis_index("c")
    sid = lax.axis_index("s")
    wid = sid * _NC + cid
    row0 = wid * rows_per_tile

    pltpu.sync_copy(perm_hbm, perm_v)

    zvec = jnp.zeros((_L,), jnp.float32)

    def _zero(i, carry):
        zv[pl.ds(i * _L, _L)] = zvec
        return carry

    lax.fori_loop(0, rows_per_tile // _L, _zero, 0)
    pltpu.sync_copy(zv, ld_hbm.at[pl.ds(row0, rows_per_tile)])

    ins = (in0, in1, in2, in3)
    outs = (out0, out1, out2, out3)
    isems = (si0, si1, si2, si3)
    osems = (so0, so1, so2, so3)

    def in_copy(c, b):
        return pltpu.make_async_copy(
            x_hbm.at[pl.ds(row0 + c * _K, _K)], ins[b], isems[b])

    def out_copy(c, p, q):
        return pltpu.make_async_copy(
            outs[q],
            y_hbm.at[pl.ds(row0 + c * _K, _K), pl.ds(p * halfn, halfn)],
            osems[q])

    for c0 in range(_NI - 1):
        in_copy(c0, c0).start()

    rvec = [jnp.full((_L,), r, jnp.int32) for r in range(_K)]

    def chunk_group(g, carry):
        for b in range(_NI):
            c = g * _NI + b

            @pl.when(c + _NI - 1 < nch)
            def _start_next():
                in_copy(c + _NI - 1, (b + _NI - 1) % _NI).start()

            in_copy(c, b).wait()
            ib = ins[b]

            for p in range(_P):
                q = (2 * b + p) % _NO

                @pl.when(c >= 2)
                def _free_out():
                    out_copy(c - 2, p, q).wait()

                ob = outs[q]
                j0 = p * nj

                @plsc.parallel_loop(0, nj, unroll=8)
                def _jbody(jj):
                    idx = perm_v[pl.ds((j0 + jj) * _L, _L)]
                    for r in range(_K):
                        ob[r, pl.ds(jj * _L, _L)] = (
                            plsc.load_gather(ib, [rvec[r], idx]))
                out_copy(c, p, q).start()
        return carry

    lax.fori_loop(0, nch // _NI, chunk_group, 0)
    for p in range(_P):
        out_copy(nch - 2, p, (2 * (_NI - 2) + p) % _NO).wait()
        out_copy(nch - 1, p, (2 * (_NI - 1) + p) % _NO).wait()


def kernel(x, perm):
    batch, n = x.shape
    perm = perm.astype(jnp.int32)
    mesh = plsc.VectorSubcoreMesh(core_axis_name="c", subcore_axis_name="s")
    call = pl.kernel(
        functools.partial(_body, batch, n),
        out_type=(
            jax.ShapeDtypeStruct((batch, n), x.dtype),
            jax.ShapeDtypeStruct((batch,), x.dtype),
        ),
        mesh=mesh,
        compiler_params=pltpu.CompilerParams(needs_layout_passes=False),
        scratch_types=[
            pltpu.VMEM((n,), jnp.int32),
            pltpu.VMEM((_K, n), jnp.float32),
            pltpu.VMEM((_K, n), jnp.float32),
            pltpu.VMEM((_K, n), jnp.float32),
            pltpu.VMEM((_K, n), jnp.float32),
            pltpu.VMEM((_K, n // _P), jnp.float32),
            pltpu.VMEM((_K, n // _P), jnp.float32),
            pltpu.VMEM((_K, n // _P), jnp.float32),
            pltpu.VMEM((_K, n // _P), jnp.float32),
            pltpu.VMEM((batch // _NW,), jnp.float32),
            pltpu.SemaphoreType.DMA,
            pltpu.SemaphoreType.DMA,
            pltpu.SemaphoreType.DMA,
            pltpu.SemaphoreType.DMA,
            pltpu.SemaphoreType.DMA,
            pltpu.SemaphoreType.DMA,
            pltpu.SemaphoreType.DMA,
            pltpu.SemaphoreType.DMA,
        ],
    )
    y, log_det = call(x, perm)
    return y, log_det
